# baseline (device time: 51268 ns/iter reference)
import jax
import jax.numpy as jnp
from jax import lax
from jax.experimental import pallas as pl
from jax.experimental.pallas import tpu as pltpu

N_DEV = 32
SQ = 256
D_MODEL = 1024
SKV = 4096
DH = 128
HQ_LOCAL = 8
KV_LOCAL = 2
ROWS = SQ // N_DEV
N_BLK = 4
BLK = SQ // N_BLK
CHUNKS_PER_BLK = BLK // ROWS
SCALE = 0.08838834764831843


def kernel(x, Wq, Wo, K_ext, V_ext):
    def body(x_ref, wq_ref, wo_hbm, k_hbm, v_hbm, out_ref,
             k_vmem, v_vmem, wo_vmem, attn_ref, rs_src, rs_buf, ag_buf,
             kv_sems, wo_sem, rs_send, rs_recv, ag_send, ag_recv):
        my_id = lax.axis_index("i")

        kv_start = KV_LOCAL * my_id
        kv_copies = []
        for h in range(KV_LOCAL):
            ck = pltpu.make_async_copy(
                k_hbm.at[0, :, kv_start + h, :], k_vmem.at[h], kv_sems.at[h, 0])
            cv = pltpu.make_async_copy(
                v_hbm.at[0, :, kv_start + h, :], v_vmem.at[h], kv_sems.at[h, 1])
            ck.start()
            cv.start()
            kv_copies.append((ck, cv))
        wo_copy = pltpu.make_async_copy(wo_hbm.at[...], wo_vmem, wo_sem)
        wo_copy.start()

        barrier = pltpu.get_barrier_semaphore()
        for d in range(1, N_DEV):
            peer = lax.rem(my_id + d, N_DEV)
            pl.semaphore_signal(barrier, inc=1, device_id=(peer,),
                                device_id_type=pl.DeviceIdType.MESH)

        xb = x_ref[0].astype(jnp.bfloat16)
        wqb = wq_ref[...].astype(jnp.bfloat16)
        q = jnp.dot(xb, wqb, preferred_element_type=jnp.float32)

        for ck, cv in kv_copies:
            ck.wait()
            cv.wait()
        ks = [k_vmem[h].astype(jnp.bfloat16) for h in range(KV_LOCAL)]
        vs = [v_vmem[h].astype(jnp.bfloat16) for h in range(KV_LOCAL)]
        wo_copy.wait()
        wob = wo_vmem[...].astype(jnp.bfloat16)

        for b in range(N_BLK):
            r0 = BLK * b
            for h in range(HQ_LOCAL):
                qh = q[r0:r0 + BLK, h * DH:(h + 1) * DH].astype(jnp.bfloat16)
                s = lax.dot_general(qh, ks[h // 4], (((1,), (1,)), ((), ())),
                                    preferred_element_type=jnp.float32) * SCALE
                p = jnp.exp(s.astype(jnp.bfloat16))
                l = jnp.sum(p, axis=1, keepdims=True, dtype=jnp.float32)
                o = jnp.dot(p, vs[h // 4],
                            preferred_element_type=jnp.float32) / l
                attn_ref[r0:r0 + BLK, h * DH:(h + 1) * DH] = o.astype(jnp.bfloat16)

            pblk = jnp.dot(attn_ref[r0:r0 + BLK, :], wob,
                           preferred_element_type=jnp.float32)
            rs_src[CHUNKS_PER_BLK * b:CHUNKS_PER_BLK * (b + 1)] = (
                pblk.astype(jnp.bfloat16).reshape(CHUNKS_PER_BLK, ROWS, D_MODEL))

            if b == 0:
                pl.semaphore_wait(barrier, N_DEV - 1)

            for c in range(CHUNKS_PER_BLK * b, CHUNKS_PER_BLK * (b + 1)):
                d = lax.rem(jnp.int32(c) - my_id + N_DEV, N_DEV)

                @pl.when(d == 0)
                def _keep(c=c):
                    rs_buf[0] = rs_src[c]

                @pl.when(d != 0)
                def _send(c=c, d=d):
                    rdma = pltpu.make_async_remote_copy(
                        src_ref=rs_src.at[c],
                        dst_ref=rs_buf.at[d],
                        send_sem=rs_send.at[d],
                        recv_sem=rs_recv.at[d],
                        device_id=(jnp.int32(c),),
                        device_id_type=pl.DeviceIdType.MESH,
                    )
                    rdma.start()

        for d in range(1, N_DEV):
            pltpu.make_async_remote_copy(
                src_ref=rs_src.at[0],
                dst_ref=rs_buf.at[d],
                send_sem=rs_send.at[0],
                recv_sem=rs_recv.at[d],
                device_id=(my_id,),
                device_id_type=pl.DeviceIdType.MESH,
            ).wait_recv()

        total = jnp.sum(rs_buf[...].astype(jnp.float32), axis=0)
        ag_buf[my_id] = total.astype(jnp.bfloat16)

        ag_rdmas = []
        for d in range(1, N_DEV):
            peer = lax.rem(my_id + d, N_DEV)
            rdma = pltpu.make_async_remote_copy(
                src_ref=ag_buf.at[my_id],
                dst_ref=ag_buf.at[my_id],
                send_sem=ag_send.at[d],
                recv_sem=ag_recv.at[d],
                device_id=(peer,),
                device_id_type=pl.DeviceIdType.MESH,
            )
            rdma.start()
            ag_rdmas.append(rdma)
        for rdma in ag_rdmas:
            rdma.wait_recv()
        out_ref[0] = ag_buf[...].astype(jnp.float32).reshape(SQ, D_MODEL)

        for d in range(1, N_DEV):
            pltpu.make_async_remote_copy(
                src_ref=rs_src.at[0],
                dst_ref=rs_buf.at[d],
                send_sem=rs_send.at[d],
                recv_sem=rs_recv.at[0],
                device_id=(my_id,),
                device_id_type=pl.DeviceIdType.MESH,
            ).wait_send()
        for rdma in ag_rdmas:
            rdma.wait_send()

    return pl.pallas_call(
        body,
        out_shape=jax.ShapeDtypeStruct((1, SQ, D_MODEL), jnp.float32),
        in_specs=[
            pl.BlockSpec(memory_space=pltpu.VMEM),
            pl.BlockSpec(memory_space=pltpu.VMEM),
            pl.BlockSpec(memory_space=pl.ANY),
            pl.BlockSpec(memory_space=pl.ANY),
            pl.BlockSpec(memory_space=pl.ANY),
        ],
        out_specs=pl.BlockSpec(memory_space=pltpu.VMEM),
        scratch_shapes=[
            pltpu.VMEM((KV_LOCAL, SKV, DH), jnp.float32),
            pltpu.VMEM((KV_LOCAL, SKV, DH), jnp.float32),
            pltpu.VMEM((D_MODEL, D_MODEL), jnp.float32),
            pltpu.VMEM((SQ, D_MODEL), jnp.bfloat16),
            pltpu.VMEM((N_DEV, ROWS, D_MODEL), jnp.bfloat16),
            pltpu.VMEM((N_DEV, ROWS, D_MODEL), jnp.bfloat16),
            pltpu.VMEM((N_DEV, ROWS, D_MODEL), jnp.bfloat16),
            pltpu.SemaphoreType.DMA((KV_LOCAL, 2)),
            pltpu.SemaphoreType.DMA(()),
            pltpu.SemaphoreType.DMA((N_DEV,)),
            pltpu.SemaphoreType.DMA((N_DEV,)),
            pltpu.SemaphoreType.DMA((N_DEV,)),
            pltpu.SemaphoreType.DMA((N_DEV,)),
        ],
        compiler_params=pltpu.CompilerParams(collective_id=0),
    )(x, Wq, Wo, K_ext, V_ext)


# device time: 41785 ns/iter; 1.2269x vs baseline; 1.2269x over previous
import jax
import jax.numpy as jnp
from jax import lax
from jax.experimental import pallas as pl
from jax.experimental.pallas import tpu as pltpu

N_DEV = 32
SQ = 256
D_MODEL = 1024
SKV = 4096
DH = 128
HQ_LOCAL = 8
KV_LOCAL = 2
ROWS = SQ // N_DEV
N_BLK = 1
BLK = SQ // N_BLK
CHUNKS_PER_BLK = BLK // ROWS
SCALE = 0.08838834764831843


def kernel(x, Wq, Wo, K_ext, V_ext):
    def body(x_ref, wq_ref, wo_hbm, k_hbm, v_hbm, out_ref,
             k_vmem, v_vmem, wo_vmem, attn_ref, rs_src, rs_buf, ag_buf,
             kv_sems, wo_sem, rs_send, rs_recv, ag_send, ag_recv):
        my_id = lax.axis_index("i")

        kv_start = KV_LOCAL * my_id
        kv_copies = []
        for h in range(KV_LOCAL):
            ck = pltpu.make_async_copy(
                k_hbm.at[0, :, kv_start + h, :], k_vmem.at[h], kv_sems.at[h, 0])
            cv = pltpu.make_async_copy(
                v_hbm.at[0, :, kv_start + h, :], v_vmem.at[h], kv_sems.at[h, 1])
            ck.start()
            cv.start()
            kv_copies.append((ck, cv))
        wo_copy = pltpu.make_async_copy(wo_hbm.at[...], wo_vmem, wo_sem)
        wo_copy.start()

        barrier = pltpu.get_barrier_semaphore()
        for d in range(1, N_DEV):
            peer = lax.rem(my_id + d, N_DEV)
            pl.semaphore_signal(barrier, inc=1, device_id=(peer,),
                                device_id_type=pl.DeviceIdType.MESH)

        xb = x_ref[0].astype(jnp.bfloat16)
        wqb = wq_ref[...].astype(jnp.bfloat16)
        q = jnp.dot(xb, wqb, preferred_element_type=jnp.float32)

        for ck, cv in kv_copies:
            ck.wait()
            cv.wait()
        ks = [k_vmem[h].astype(jnp.bfloat16) for h in range(KV_LOCAL)]
        vs = [v_vmem[h].astype(jnp.bfloat16) for h in range(KV_LOCAL)]
        wo_copy.wait()
        wob = wo_vmem[...].astype(jnp.bfloat16)

        for b in range(N_BLK):
            r0 = BLK * b
            for h in range(HQ_LOCAL):
                qh = q[r0:r0 + BLK, h * DH:(h + 1) * DH].astype(jnp.bfloat16)
                s = lax.dot_general(qh, ks[h // 4], (((1,), (1,)), ((), ())),
                                    preferred_element_type=jnp.float32) * SCALE
                p = jnp.exp(s.astype(jnp.bfloat16))
                l = jnp.sum(p, axis=1, keepdims=True, dtype=jnp.float32)
                o = jnp.dot(p, vs[h // 4],
                            preferred_element_type=jnp.float32) / l
                attn_ref[r0:r0 + BLK, h * DH:(h + 1) * DH] = o.astype(jnp.bfloat16)

            pblk = jnp.dot(attn_ref[r0:r0 + BLK, :], wob,
                           preferred_element_type=jnp.float32)
            rs_src[CHUNKS_PER_BLK * b:CHUNKS_PER_BLK * (b + 1)] = (
                pblk.astype(jnp.bfloat16).reshape(CHUNKS_PER_BLK, ROWS, D_MODEL))

            if b == 0:
                pl.semaphore_wait(barrier, N_DEV - 1)

            for c in range(CHUNKS_PER_BLK * b, CHUNKS_PER_BLK * (b + 1)):
                d = lax.rem(jnp.int32(c) - my_id + N_DEV, N_DEV)

                @pl.when(d == 0)
                def _keep(c=c):
                    rs_buf[0] = rs_src[c]

                @pl.when(d != 0)
                def _send(c=c, d=d):
                    rdma = pltpu.make_async_remote_copy(
                        src_ref=rs_src.at[c],
                        dst_ref=rs_buf.at[d],
                        send_sem=rs_send.at[d],
                        recv_sem=rs_recv.at[d],
                        device_id=(jnp.int32(c),),
                        device_id_type=pl.DeviceIdType.MESH,
                    )
                    rdma.start()

        for d in range(1, N_DEV):
            pltpu.make_async_remote_copy(
                src_ref=rs_src.at[0],
                dst_ref=rs_buf.at[d],
                send_sem=rs_send.at[0],
                recv_sem=rs_recv.at[d],
                device_id=(my_id,),
                device_id_type=pl.DeviceIdType.MESH,
            ).wait_recv()

        total = jnp.sum(rs_buf[...].astype(jnp.float32), axis=0)
        ag_buf[my_id] = total.astype(jnp.bfloat16)

        ag_rdmas = []
        for d in range(1, N_DEV):
            peer = lax.rem(my_id + d, N_DEV)
            rdma = pltpu.make_async_remote_copy(
                src_ref=ag_buf.at[my_id],
                dst_ref=ag_buf.at[my_id],
                send_sem=ag_send.at[d],
                recv_sem=ag_recv.at[d],
                device_id=(peer,),
                device_id_type=pl.DeviceIdType.MESH,
            )
            rdma.start()
            ag_rdmas.append(rdma)
        for rdma in ag_rdmas:
            rdma.wait_recv()
        out_ref[0] = ag_buf[...].astype(jnp.float32).reshape(SQ, D_MODEL)

        for d in range(1, N_DEV):
            pltpu.make_async_remote_copy(
                src_ref=rs_src.at[0],
                dst_ref=rs_buf.at[d],
                send_sem=rs_send.at[d],
                recv_sem=rs_recv.at[0],
                device_id=(my_id,),
                device_id_type=pl.DeviceIdType.MESH,
            ).wait_send()
        for rdma in ag_rdmas:
            rdma.wait_send()

    return pl.pallas_call(
        body,
        out_shape=jax.ShapeDtypeStruct((1, SQ, D_MODEL), jnp.float32),
        in_specs=[
            pl.BlockSpec(memory_space=pltpu.VMEM),
            pl.BlockSpec(memory_space=pltpu.VMEM),
            pl.BlockSpec(memory_space=pl.ANY),
            pl.BlockSpec(memory_space=pl.ANY),
            pl.BlockSpec(memory_space=pl.ANY),
        ],
        out_specs=pl.BlockSpec(memory_space=pltpu.VMEM),
        scratch_shapes=[
            pltpu.VMEM((KV_LOCAL, SKV, DH), jnp.float32),
            pltpu.VMEM((KV_LOCAL, SKV, DH), jnp.float32),
            pltpu.VMEM((D_MODEL, D_MODEL), jnp.float32),
            pltpu.VMEM((SQ, D_MODEL), jnp.bfloat16),
            pltpu.VMEM((N_DEV, ROWS, D_MODEL), jnp.bfloat16),
            pltpu.VMEM((N_DEV, ROWS, D_MODEL), jnp.bfloat16),
            pltpu.VMEM((N_DEV, ROWS, D_MODEL), jnp.bfloat16),
            pltpu.SemaphoreType.DMA((KV_LOCAL, 2)),
            pltpu.SemaphoreType.DMA(()),
            pltpu.SemaphoreType.DMA((N_DEV,)),
            pltpu.SemaphoreType.DMA((N_DEV,)),
            pltpu.SemaphoreType.DMA((N_DEV,)),
            pltpu.SemaphoreType.DMA((N_DEV,)),
        ],
        compiler_params=pltpu.CompilerParams(collective_id=0),
    )(x, Wq, Wo, K_ext, V_ext)


# device time: 41352 ns/iter; 1.2398x vs baseline; 1.0105x over previous
import jax
import jax.numpy as jnp
from jax import lax
from jax.experimental import pallas as pl
from jax.experimental.pallas import tpu as pltpu

N_DEV = 32
SQ = 256
D_MODEL = 1024
SKV = 4096
DH = 128
HQ_LOCAL = 8
KV_LOCAL = 2
ROWS = SQ // N_DEV
N_BLK = 1
BLK = SQ // N_BLK
CHUNKS_PER_BLK = BLK // ROWS
SCALE = 0.08838834764831843


def kernel(x, Wq, Wo, K_ext, V_ext):
    def body(x_ref, wq_ref, wo_hbm, k_hbm, v_hbm, out_ref,
             k_vmem, v_vmem, wo_vmem, attn_ref, rs_src, rs_buf, ag_buf,
             kv_sems, wo_sem, rs_send, rs_recv, ag_send, ag_recv):
        my_id = lax.axis_index("i")

        kv_start = KV_LOCAL * my_id
        kv_copies = []
        for h in range(KV_LOCAL):
            ck = pltpu.make_async_copy(
                k_hbm.at[0, :, kv_start + h, :], k_vmem.at[h], kv_sems.at[h, 0])
            cv = pltpu.make_async_copy(
                v_hbm.at[0, :, kv_start + h, :], v_vmem.at[h], kv_sems.at[h, 1])
            ck.start()
            cv.start()
            kv_copies.append((ck, cv))
        wo_copy = pltpu.make_async_copy(wo_hbm.at[...], wo_vmem, wo_sem)
        wo_copy.start()

        barrier = pltpu.get_barrier_semaphore()
        for d in range(1, N_DEV):
            peer = lax.rem(my_id + d, N_DEV)
            pl.semaphore_signal(barrier, inc=1, device_id=(peer,),
                                device_id_type=pl.DeviceIdType.MESH)

        xb = x_ref[0].astype(jnp.bfloat16)
        wqb = wq_ref[...].astype(jnp.bfloat16)
        q = jnp.dot(xb, wqb, preferred_element_type=jnp.float32) * SCALE

        for ck, cv in kv_copies:
            ck.wait()
            cv.wait()
        ks = [k_vmem[h].astype(jnp.bfloat16) for h in range(KV_LOCAL)]
        vs = [v_vmem[h].astype(jnp.bfloat16) for h in range(KV_LOCAL)]
        wo_copy.wait()
        wob = wo_vmem[...].astype(jnp.bfloat16)

        for b in range(N_BLK):
            r0 = BLK * b
            for h in range(HQ_LOCAL):
                qh = q[r0:r0 + BLK, h * DH:(h + 1) * DH].astype(jnp.bfloat16)
                s = lax.dot_general(qh, ks[h // 4], (((1,), (1,)), ((), ())),
                                    preferred_element_type=jnp.float32)
                p = jnp.exp(s.astype(jnp.bfloat16))
                l = jnp.sum(p, axis=1, keepdims=True, dtype=jnp.float32)
                o = jnp.dot(p, vs[h // 4],
                            preferred_element_type=jnp.float32) / l
                attn_ref[r0:r0 + BLK, h * DH:(h + 1) * DH] = o.astype(jnp.bfloat16)

            pblk = jnp.dot(attn_ref[r0:r0 + BLK, :], wob,
                           preferred_element_type=jnp.float32)
            rs_src[CHUNKS_PER_BLK * b:CHUNKS_PER_BLK * (b + 1)] = (
                pblk.astype(jnp.bfloat16).reshape(CHUNKS_PER_BLK, ROWS, D_MODEL))

            if b == 0:
                pl.semaphore_wait(barrier, N_DEV - 1)

            for c in range(CHUNKS_PER_BLK * b, CHUNKS_PER_BLK * (b + 1)):
                d = lax.rem(jnp.int32(c) - my_id + N_DEV, N_DEV)

                @pl.when(d == 0)
                def _keep(c=c):
                    rs_buf[0] = rs_src[c]

                @pl.when(d != 0)
                def _send(c=c, d=d):
                    rdma = pltpu.make_async_remote_copy(
                        src_ref=rs_src.at[c],
                        dst_ref=rs_buf.at[d],
                        send_sem=rs_send.at[d],
                        recv_sem=rs_recv.at[d],
                        device_id=(jnp.int32(c),),
                        device_id_type=pl.DeviceIdType.MESH,
                    )
                    rdma.start()

        for d in range(1, N_DEV):
            pltpu.make_async_remote_copy(
                src_ref=rs_src.at[0],
                dst_ref=rs_buf.at[d],
                send_sem=rs_send.at[0],
                recv_sem=rs_recv.at[d],
                device_id=(my_id,),
                device_id_type=pl.DeviceIdType.MESH,
            ).wait_recv()

        total = jnp.sum(rs_buf[...].astype(jnp.float32), axis=0)
        ag_buf[my_id] = total.astype(jnp.bfloat16)

        ag_rdmas = []
        for d in range(1, N_DEV):
            peer = lax.rem(my_id + d, N_DEV)
            rdma = pltpu.make_async_remote_copy(
                src_ref=ag_buf.at[my_id],
                dst_ref=ag_buf.at[my_id],
                send_sem=ag_send.at[d],
                recv_sem=ag_recv.at[d],
                device_id=(peer,),
                device_id_type=pl.DeviceIdType.MESH,
            )
            rdma.start()
            ag_rdmas.append(rdma)
        out_ref[0, pl.ds(my_id * ROWS, ROWS), :] = total
        for d, rdma in zip(range(1, N_DEV), ag_rdmas):
            rdma.wait_recv()
            origin = lax.rem(my_id - d + N_DEV, N_DEV)
            out_ref[0, pl.ds(origin * ROWS, ROWS), :] = (
                ag_buf[origin].astype(jnp.float32))

        for d in range(1, N_DEV):
            pltpu.make_async_remote_copy(
                src_ref=rs_src.at[0],
                dst_ref=rs_buf.at[d],
                send_sem=rs_send.at[d],
                recv_sem=rs_recv.at[0],
                device_id=(my_id,),
                device_id_type=pl.DeviceIdType.MESH,
            ).wait_send()
        for rdma in ag_rdmas:
            rdma.wait_send()

    return pl.pallas_call(
        body,
        out_shape=jax.ShapeDtypeStruct((1, SQ, D_MODEL), jnp.float32),
        in_specs=[
            pl.BlockSpec(memory_space=pltpu.VMEM),
            pl.BlockSpec(memory_space=pltpu.VMEM),
            pl.BlockSpec(memory_space=pl.ANY),
            pl.BlockSpec(memory_space=pl.ANY),
            pl.BlockSpec(memory_space=pl.ANY),
        ],
        out_specs=pl.BlockSpec(memory_space=pltpu.VMEM),
        scratch_shapes=[
            pltpu.VMEM((KV_LOCAL, SKV, DH), jnp.float32),
            pltpu.VMEM((KV_LOCAL, SKV, DH), jnp.float32),
            pltpu.VMEM((D_MODEL, D_MODEL), jnp.float32),
            pltpu.VMEM((SQ, D_MODEL), jnp.bfloat16),
            pltpu.VMEM((N_DEV, ROWS, D_MODEL), jnp.bfloat16),
            pltpu.VMEM((N_DEV, ROWS, D_MODEL), jnp.bfloat16),
            pltpu.VMEM((N_DEV, ROWS, D_MODEL), jnp.bfloat16),
            pltpu.SemaphoreType.DMA((KV_LOCAL, 2)),
            pltpu.SemaphoreType.DMA(()),
            pltpu.SemaphoreType.DMA((N_DEV,)),
            pltpu.SemaphoreType.DMA((N_DEV,)),
            pltpu.SemaphoreType.DMA((N_DEV,)),
            pltpu.SemaphoreType.DMA((N_DEV,)),
        ],
        compiler_params=pltpu.CompilerParams(collective_id=0),
    )(x, Wq, Wo, K_ext, V_ext)
